# Initial kernel scaffold; baseline (speedup 1.0000x reference)
import jax
import jax.numpy as jnp
from jax.experimental import pallas as pl


def kernel(x, edge_index, batch, W1, a_src1, a_dst1, b1, g1, be1, W2, a_src2, a_dst2, b2, g2, be2, fc1_w, fc1_b, fc2_w, fc2_b, fc3_w, fc3_b):
    raise NotImplementedError("write your pallas kernel here")


# speedup vs baseline: 23.2477x; 23.2477x over previous
import dataclasses
import functools

import jax
import jax.numpy as jnp
from jax import lax
from jax.experimental import pallas as pl
from jax.experimental.pallas import tpu as pltpu
from jax.experimental.pallas import tpu_sc as plsc

N = 10000
NP = 10240
E = 320000
G = 64

NC = 2
NS = 16
NW = NC * NS
K = 128
NSUB = 81
CHUNK = NSUB * K
EPAD = NW * CHUNK
ETOT = E + N

RPT = NP // NS

BN_BLK = 1024
NBLK = NP // BN_BLK


DH2 = 64


def _gat_edge_call(nh):
  mesh = plsc.VectorSubcoreMesh(
      core_axis_name="c", subcore_axis_name="s", num_cores=NC, num_subcores=NS)
  cp = pltpu.CompilerParams()
  for fld, val in (("needs_layout_passes", False),
                   ("use_tc_tiling_on_sc", False)):
    if fld in pltpu.CompilerParams.__dataclass_fields__:
      cp = dataclasses.replace(cp, **{fld: val})

  @functools.partial(
      pl.kernel,
      compiler_params=cp,
      out_type=(jax.ShapeDtypeStruct((nh, NC, NP, DH2), jnp.float32),
                jax.ShapeDtypeStruct((NC, NP), jnp.float32)),
      mesh=mesh,
      scratch_types=[
          pltpu.VMEM((NP,), jnp.float32),
          pltpu.VMEM((NP,), jnp.float32),
          pltpu.VMEM((NSUB, K), jnp.int32),
          pltpu.VMEM((NSUB, K), jnp.int32),
          pltpu.VMEM((NSUB, K), jnp.float32),
          pltpu.VMEM((K, DH2), jnp.float32),
          pltpu.VMEM((RPT,), jnp.float32),
          pltpu.VMEM_SHARED((NP, DH2), jnp.float32),
          pltpu.VMEM_SHARED((NP,), jnp.float32),
      ],
  )
  def k(*refs):
    h_hbms = refs[0:nh]
    asrc_hbm, adst_hbm, src_hbm, dst_hbm = refs[nh:nh + 4]
    acc_hbm, den_hbm = refs[nh + 4:nh + 6]
    (asrc_v, adst_v, src_v, dst_v, ex_v, rows_v, tmp1d_v,
     acc_s, den_s) = refs[nh + 6:]

    c = lax.axis_index("c")
    s = lax.axis_index("s")
    wid = c * NS + s
    ebase = wid * CHUNK
    zf = jnp.zeros((16,), jnp.float32)

    pltpu.sync_copy(asrc_hbm, asrc_v)
    pltpu.sync_copy(adst_hbm, adst_v)
    pltpu.sync_copy(src_hbm.at[wid], src_v)
    pltpu.sync_copy(dst_hbm.at[wid], dst_v)

    @pl.loop(0, RPT // 16)
    def _(t):
      tmp1d_v[pl.ds(t * 16, 16)] = zf

    for hf in range(nh):
      @pl.loop(0, K)
      def _(r):
        for q in range(DH2 // 16):
          rows_v[r, pl.ds(q * 16, 16)] = zf

      for q in range(RPT // K):
        pltpu.sync_copy(rows_v, acc_s.at[pl.ds(s * RPT + q * K, K)])
      if hf == 0:
        pltpu.sync_copy(tmp1d_v, den_s.at[pl.ds(s * RPT, RPT)])

      plsc.subcore_barrier()

      @pl.loop(0, NSUB)
      def _(j):
        if hf == 0:
          @pl.loop(0, K // 16)
          def _(i):
            sidx = src_v[j, pl.ds(i * 16, 16)]
            didx = dst_v[j, pl.ds(i * 16, 16)]
            vs = plsc.load_gather(asrc_v, [sidx])
            vd = plsc.load_gather(adst_v, [didx])
            e = vs + vd
            e = jnp.where(e >= 0.0, e, 0.2 * e)
            ex = jnp.exp(e)
            gi = ebase + j * K + i * 16 + lax.iota(jnp.int32, 16)
            ex = jnp.where(gi < ETOT, ex, 0.0)
            ex_v[j, pl.ds(i * 16, 16)] = ex

          pltpu.sync_copy(ex_v.at[j], den_s.at[dst_v.at[j]], add=True)

        pltpu.sync_copy(h_hbms[hf].at[src_v.at[j]], rows_v)

        @pl.loop(0, K)
        def _(i):
          jv = jnp.full((16,), j, jnp.int32)
          iv = jnp.full((16,), i, jnp.int32)
          w = plsc.load_gather(ex_v, [jv, iv])
          for q in range(DH2 // 16):
            sl = pl.ds(q * 16, 16)
            rows_v[i, sl] = rows_v[i, sl] * w

        pltpu.sync_copy(rows_v, acc_s.at[dst_v.at[j]], add=True)

      plsc.subcore_barrier()

      for q in range(RPT // K):
        r0 = s * RPT + q * K
        pltpu.sync_copy(acc_s.at[pl.ds(r0, K)], rows_v)
        pltpu.sync_copy(rows_v, acc_hbm.at[hf, c, pl.ds(r0, K)])
      if hf == 0:
        pltpu.sync_copy(den_s.at[pl.ds(s * RPT, RPT)], tmp1d_v)
        pltpu.sync_copy(tmp1d_v, den_hbm.at[c, pl.ds(s * RPT, RPT)])

      plsc.subcore_barrier()

  return k


def _project(x, W, a_s, a_d):
  Din = x.shape[1]
  D = W.shape[1]

  def body(x_ref, w_ref, as_ref, ad_ref, ha_ref, hb_ref, pas_ref, pad_ref):
    i = pl.program_id(0)
    h = jnp.dot(x_ref[...], w_ref[...], preferred_element_type=jnp.float32)
    ha_ref[...] = h[:, :DH2]
    hb_ref[...] = h[:, DH2:]
    sl = pl.ds(i * BN_BLK, BN_BLK)
    pas_ref[sl] = jnp.dot(h, as_ref[...], preferred_element_type=jnp.float32)
    pad_ref[sl] = jnp.dot(h, ad_ref[...], preferred_element_type=jnp.float32)

  return pl.pallas_call(
      body,
      grid=(NBLK,),
      in_specs=[
          pl.BlockSpec((BN_BLK, Din), lambda i: (i, 0)),
          pl.BlockSpec((Din, D), lambda i: (0, 0)),
          pl.BlockSpec((D,), lambda i: (0,)),
          pl.BlockSpec((D,), lambda i: (0,)),
      ],
      out_specs=[
          pl.BlockSpec((BN_BLK, DH2), lambda i: (i, 0)),
          pl.BlockSpec((BN_BLK, DH2), lambda i: (i, 0)),
          pl.BlockSpec((NP,), lambda i: (0,)),
          pl.BlockSpec((NP,), lambda i: (0,)),
      ],
      out_shape=[
          jax.ShapeDtypeStruct((NP, DH2), jnp.float32),
          jax.ShapeDtypeStruct((NP, DH2), jnp.float32),
          jax.ShapeDtypeStruct((NP,), jnp.float32),
          jax.ShapeDtypeStruct((NP,), jnp.float32),
      ],
  )(x, W, a_s, a_d)


def _merge(acc, den_t):
  D = acc.shape[2]

  def body(a0_ref, a1_ref, dn_ref, o_ref, st_ref):
    i = pl.program_id(0)
    o = a0_ref[...] + a1_ref[...]
    dn = dn_ref[...]
    dsum = dn[:, 0:1] + dn[:, 1:2]
    o = o / (dsum + 1e-16)
    o_ref[...] = o

    @pl.when(i == 0)
    def _():
      st_ref[...] = jnp.zeros_like(st_ref)

    st_ref[0:1, :] += jnp.sum(o, axis=0, keepdims=True)
    st_ref[1:2, :] += jnp.sum(o * o, axis=0, keepdims=True)

  return pl.pallas_call(
      body,
      grid=(NBLK,),
      in_specs=[
          pl.BlockSpec((BN_BLK, D), lambda i: (i, 0)),
          pl.BlockSpec((BN_BLK, D), lambda i: (i, 0)),
          pl.BlockSpec((BN_BLK, NC), lambda i: (i, 0)),
      ],
      out_specs=[
          pl.BlockSpec((BN_BLK, D), lambda i: (i, 0)),
          pl.BlockSpec((8, D), lambda i: (0, 0)),
      ],
      out_shape=[
          jax.ShapeDtypeStruct((NP, D), jnp.float32),
          jax.ShapeDtypeStruct((8, D), jnp.float32),
      ],
  )(acc[0], acc[1], den_t)


def _merge2(accA, accB, den_t):
  def body(aa0_ref, aa1_ref, ab0_ref, ab1_ref, dn_ref, o_ref, st_ref):
    i = pl.program_id(0)
    dn = dn_ref[...]
    dsum = dn[:, 0:1] + dn[:, 1:2] + 1e-16
    oa = (aa0_ref[...] + aa1_ref[...]) / dsum
    ob = (ab0_ref[...] + ab1_ref[...]) / dsum
    o = jnp.concatenate([oa, ob], axis=1)
    o_ref[...] = o

    @pl.when(i == 0)
    def _():
      st_ref[...] = jnp.zeros_like(st_ref)

    st_ref[0:1, :] += jnp.sum(o, axis=0, keepdims=True)
    st_ref[1:2, :] += jnp.sum(o * o, axis=0, keepdims=True)

  return pl.pallas_call(
      body,
      grid=(NBLK,),
      in_specs=[
          pl.BlockSpec((BN_BLK, DH2), lambda i: (i, 0)),
          pl.BlockSpec((BN_BLK, DH2), lambda i: (i, 0)),
          pl.BlockSpec((BN_BLK, DH2), lambda i: (i, 0)),
          pl.BlockSpec((BN_BLK, DH2), lambda i: (i, 0)),
          pl.BlockSpec((BN_BLK, NC), lambda i: (i, 0)),
      ],
      out_specs=[
          pl.BlockSpec((BN_BLK, 2 * DH2), lambda i: (i, 0)),
          pl.BlockSpec((8, 2 * DH2), lambda i: (0, 0)),
      ],
      out_shape=[
          jax.ShapeDtypeStruct((NP, 2 * DH2), jnp.float32),
          jax.ShapeDtypeStruct((8, 2 * DH2), jnp.float32),
      ],
  )(accA[0], accA[1], accB[0], accB[1], den_t)


def _normproj(o, st, g, be, W, a_s, a_d):
  D = o.shape[1]
  DO = W.shape[1]

  def body(o_ref, st_ref, g_ref, be_ref, w_ref, as_ref, ad_ref,
           h_ref, pas_ref, pad_ref):
    st = st_ref[...]
    mu = st[0:1, :] / N
    var = st[1:2, :] / N - mu * mu
    inv = lax.rsqrt(var + 1e-5)
    xn = g_ref[...][None, :] * (o_ref[...] - mu) * inv + be_ref[...][None, :]
    xr = jnp.maximum(xn, 0.0)
    h = jnp.dot(xr, w_ref[...], preferred_element_type=jnp.float32)
    h_ref[...] = h
    i = pl.program_id(0)
    sl = pl.ds(i * BN_BLK, BN_BLK)
    pas_ref[sl] = jnp.dot(h, as_ref[...], preferred_element_type=jnp.float32)
    pad_ref[sl] = jnp.dot(h, ad_ref[...], preferred_element_type=jnp.float32)

  return pl.pallas_call(
      body,
      grid=(NBLK,),
      in_specs=[
          pl.BlockSpec((BN_BLK, D), lambda i: (i, 0)),
          pl.BlockSpec((8, D), lambda i: (0, 0)),
          pl.BlockSpec((D,), lambda i: (0,)),
          pl.BlockSpec((D,), lambda i: (0,)),
          pl.BlockSpec((D, DO), lambda i: (0, 0)),
          pl.BlockSpec((DO,), lambda i: (0,)),
          pl.BlockSpec((DO,), lambda i: (0,)),
      ],
      out_specs=[
          pl.BlockSpec((BN_BLK, DO), lambda i: (i, 0)),
          pl.BlockSpec((NP,), lambda i: (0,)),
          pl.BlockSpec((NP,), lambda i: (0,)),
      ],
      out_shape=[
          jax.ShapeDtypeStruct((NP, DO), jnp.float32),
          jax.ShapeDtypeStruct((NP,), jnp.float32),
          jax.ShapeDtypeStruct((NP,), jnp.float32),
      ],
  )(o, st, g, be, W, a_s, a_d)


def _tail(o, st, g, be, batch, f1w, f1b, f2w, f2b, f3w, f3b):
  D = o.shape[1]
  DH = f1w.shape[1]

  def body(o_ref, st_ref, g_ref, be_ref, bt_ref,
           f1w_ref, f1b_ref, f2w_ref, f2b_ref, f3w_ref, f3b_ref,
           z_ref, psum, pcnt):
    i = pl.program_id(0)
    st = st_ref[...]
    mu = st[0:1, :] / N
    var = st[1:2, :] / N - mu * mu
    inv = lax.rsqrt(var + 1e-5)
    xn = g_ref[...][None, :] * (o_ref[...] - mu) * inv + be_ref[...][None, :]
    h = jnp.maximum(xn, 0.0)

    bt = bt_ref[pl.ds(i * BN_BLK, BN_BLK)]
    oh = (bt[:, None] == lax.broadcasted_iota(jnp.int32, (1, G), 1)
          ).astype(jnp.float32)

    @pl.when(i == 0)
    def _():
      psum[...] = jnp.zeros_like(psum)
      pcnt[...] = jnp.zeros_like(pcnt)

    psum[...] += lax.dot_general(oh, h, (((0,), (0,)), ((), ())),
                                 preferred_element_type=jnp.float32)
    ones = jnp.ones((BN_BLK, 1), jnp.float32)
    pcnt[...] += lax.dot_general(oh, ones, (((0,), (0,)), ((), ())),
                                 preferred_element_type=jnp.float32)

    @pl.when(i == NBLK - 1)
    def _():
      pooled = psum[...] / jnp.maximum(pcnt[...], 1.0)
      z = jnp.maximum(
          jnp.dot(pooled, f1w_ref[...], preferred_element_type=jnp.float32)
          + f1b_ref[...][None, :], 0.0)
      z = jnp.maximum(
          jnp.dot(z, f2w_ref[...], preferred_element_type=jnp.float32)
          + f2b_ref[...][None, :], 0.0)
      z = (jnp.dot(z, f3w_ref[...], preferred_element_type=jnp.float32)
           + f3b_ref[...][None, :])
      z_ref[...] = z

  return pl.pallas_call(
      body,
      grid=(NBLK,),
      in_specs=[
          pl.BlockSpec((BN_BLK, D), lambda i: (i, 0)),
          pl.BlockSpec((8, D), lambda i: (0, 0)),
          pl.BlockSpec((D,), lambda i: (0,)),
          pl.BlockSpec((D,), lambda i: (0,)),
          pl.BlockSpec((NP,), lambda i: (0,)),
          pl.BlockSpec((D, DH), lambda i: (0, 0)),
          pl.BlockSpec((DH,), lambda i: (0,)),
          pl.BlockSpec((DH, DH // 2), lambda i: (0, 0)),
          pl.BlockSpec((DH // 2,), lambda i: (0,)),
          pl.BlockSpec((DH // 2, 1), lambda i: (0, 0)),
          pl.BlockSpec((1,), lambda i: (0,)),
      ],
      out_specs=pl.BlockSpec((G, 1), lambda i: (0, 0)),
      out_shape=jax.ShapeDtypeStruct((G, 1), jnp.float32),
      scratch_shapes=[
          pltpu.VMEM((G, D), jnp.float32),
          pltpu.VMEM((G, 1), jnp.float32),
      ],
  )(o, st, g, be, batch, f1w, f1b, f2w, f2b, f3w, f3b)


def kernel(x, edge_index, batch, W1, a_src1, a_dst1, b1, g1, be1,
           W2, a_src2, a_dst2, b2, g2, be2,
           fc1_w, fc1_b, fc2_w, fc2_b, fc3_w, fc3_b):
  loop = jnp.arange(N, dtype=edge_index.dtype)
  pad = jnp.zeros((EPAD - ETOT,), dtype=edge_index.dtype)
  src3 = jnp.concatenate([edge_index[0], loop, pad]).reshape(NW, NSUB, K)
  dst3 = jnp.concatenate([edge_index[1], loop, pad]).reshape(NW, NSUB, K)

  x_p = jnp.concatenate(
      [x, jnp.zeros((NP - N, x.shape[1]), x.dtype)], axis=0)
  batch_p = jnp.concatenate(
      [batch, jnp.full((NP - N,), G, batch.dtype)], axis=0)

  gat1 = _gat_edge_call(2)
  gat2 = _gat_edge_call(1)

  h1a, h1b, as1, ad1 = _project(x_p, W1, a_src1, a_dst1)
  acc1, den1 = gat1(h1a, h1b, as1, ad1, src3, dst3)
  o1, st1 = _merge2(acc1[0], acc1[1], den1.T)
  h2, as2, ad2 = _normproj(o1, st1, g1, be1, W2, a_src2, a_dst2)
  acc2, den2 = gat2(h2, as2, ad2, src3, dst3)
  o2, st2 = _merge(acc2[0], den2.T)
  z = _tail(o2, st2, g2, be2, batch_p, fc1_w, fc1_b, fc2_w, fc2_b, fc3_w,
            fc3_b)
  return z.reshape(G)


# speedup vs baseline: 23.5288x; 1.0121x over previous
import dataclasses
import functools

import jax
import jax.numpy as jnp
from jax import lax
from jax.experimental import pallas as pl
from jax.experimental.pallas import tpu as pltpu
from jax.experimental.pallas import tpu_sc as plsc

N = 10000
NP = 10240
E = 320000
G = 64

NC = 2
NS = 16
NW = NC * NS
K = 64
NSUB = 164
CHUNK = NSUB * K
EPAD = NW * CHUNK
ETOT = E + N

RPT = NP // NS

BN_BLK = 1024
NBLK = NP // BN_BLK


DH2 = 64


def _gat_edge_call(nh):
  mesh = plsc.VectorSubcoreMesh(
      core_axis_name="c", subcore_axis_name="s", num_cores=NC, num_subcores=NS)
  cp = pltpu.CompilerParams()
  for fld, val in (("needs_layout_passes", False),
                   ("use_tc_tiling_on_sc", False)):
    if fld in pltpu.CompilerParams.__dataclass_fields__:
      cp = dataclasses.replace(cp, **{fld: val})

  @functools.partial(
      pl.kernel,
      compiler_params=cp,
      out_type=(jax.ShapeDtypeStruct((nh, NC, NP, DH2), jnp.float32),
                jax.ShapeDtypeStruct((NC, NP), jnp.float32)),
      mesh=mesh,
      scratch_types=[
          pltpu.VMEM((NP,), jnp.float32),
          pltpu.VMEM((NP,), jnp.float32),
          pltpu.VMEM((NSUB, K), jnp.int32),
          pltpu.VMEM((NSUB, K), jnp.int32),
          pltpu.VMEM((NSUB, K), jnp.float32),
          pltpu.VMEM((K, DH2), jnp.float32),
          pltpu.VMEM((K, DH2), jnp.float32),
          pltpu.VMEM((RPT,), jnp.float32),
          pltpu.VMEM_SHARED((NP, DH2), jnp.float32),
          pltpu.VMEM_SHARED((NP,), jnp.float32),
          pltpu.SemaphoreType.DMA,
          pltpu.SemaphoreType.DMA,
          pltpu.SemaphoreType.DMA,
          pltpu.SemaphoreType.DMA,
          pltpu.SemaphoreType.DMA,
      ],
  )
  def k(*refs):
    h_hbms = refs[0:nh]
    asrc_hbm, adst_hbm, src_hbm, dst_hbm = refs[nh:nh + 4]
    acc_hbm, den_hbm = refs[nh + 4:nh + 6]
    (asrc_v, adst_v, src_v, dst_v, ex_v, rows0, rows1, tmp1d_v,
     acc_s, den_s, g0, g1, s0, s1, dsem) = refs[nh + 6:]

    c = lax.axis_index("c")
    s = lax.axis_index("s")
    wid = c * NS + s
    ebase = wid * CHUNK
    zf = jnp.zeros((16,), jnp.float32)
    rows = (rows0, rows1)
    gsem = (g0, g1)
    ssem = (s0, s1)

    pltpu.sync_copy(asrc_hbm, asrc_v)
    pltpu.sync_copy(adst_hbm, adst_v)
    pltpu.sync_copy(src_hbm.at[wid], src_v)
    pltpu.sync_copy(dst_hbm.at[wid], dst_v)

    @pl.loop(0, RPT // 16)
    def _(t):
      tmp1d_v[pl.ds(t * 16, 16)] = zf

    def prep(j, hf):
      if hf == 0:
        for i in range(K // 16):
          sl = pl.ds(i * 16, 16)
          sv = src_v[j, sl]
          dv = dst_v[j, sl]
          vs = plsc.load_gather(asrc_v, [sv])
          vd = plsc.load_gather(adst_v, [dv])
          e = vs + vd
          e = jnp.where(e >= 0.0, e, 0.2 * e)
          ex = jnp.exp(e)
          gi = ebase + j * K + i * 16 + lax.iota(jnp.int32, 16)
          ex = jnp.where(gi < ETOT, ex, 0.0)
          ex_v[j, sl] = ex
        pltpu.async_copy(ex_v.at[j], den_s.at[dst_v.at[j]], dsem, add=True)

    def gather_start(j, p, hf):
      pltpu.async_copy(h_hbms[hf].at[src_v.at[j]], rows[p], gsem[p])

    def gather_wait(j, p, hf):
      pltpu.make_async_copy(
          h_hbms[hf].at[src_v.at[j]], rows[p], gsem[p]).wait()

    def scale(j, p):
      @pl.loop(0, K, unroll=8)
      def _(i):
        jv = jnp.full((16,), j, jnp.int32)
        iv = jnp.full((16,), i, jnp.int32)
        w = plsc.load_gather(ex_v, [jv, iv])
        for q in range(DH2 // 16):
          sl = pl.ds(q * 16, 16)
          rows[p][i, sl] = rows[p][i, sl] * w

    def scatter_start(j, p):
      pltpu.async_copy(rows[p], acc_s.at[dst_v.at[j]], ssem[p], add=True)

    def scatter_wait(j, p):
      pltpu.make_async_copy(
          rows[p], acc_s.at[dst_v.at[j]], ssem[p]).wait()

    for hf in range(nh):
      @pl.loop(0, K)
      def _(r):
        for q in range(DH2 // 16):
          rows0[r, pl.ds(q * 16, 16)] = zf

      for q in range(RPT // K):
        pltpu.sync_copy(rows0, acc_s.at[pl.ds(s * RPT + q * K, K)])
      if hf == 0:
        pltpu.sync_copy(tmp1d_v, den_s.at[pl.ds(s * RPT, RPT)])

      plsc.subcore_barrier()

      prep(0, hf)
      gather_start(0, 0, hf)

      @pl.loop(0, NSUB // 2)
      def _(t):
        j0 = 2 * t
        j1 = j0 + 1

        @pl.when(t > 0)
        def _():
          scatter_wait(j1 - 2, 1)
        prep(j1, hf)
        gather_start(j1, 1, hf)

        gather_wait(j0, 0, hf)
        scale(j0, 0)
        scatter_start(j0, 0)

        @pl.when(t < NSUB // 2 - 1)
        def _():
          scatter_wait(j0, 0)
          prep(j0 + 2, hf)
          gather_start(j0 + 2, 0, hf)

        gather_wait(j1, 1, hf)
        scale(j1, 1)
        scatter_start(j1, 1)

      scatter_wait(NSUB - 2, 0)
      scatter_wait(NSUB - 1, 1)
      if hf == 0:
        @pl.loop(0, NSUB)
        def _(t):
          pltpu.make_async_copy(
              ex_v.at[0], den_s.at[dst_v.at[0]], dsem).wait()

      plsc.subcore_barrier()

      for q in range(RPT // K):
        r0 = s * RPT + q * K
        pltpu.sync_copy(acc_s.at[pl.ds(r0, K)], rows0)
        pltpu.sync_copy(rows0, acc_hbm.at[hf, c, pl.ds(r0, K)])
      if hf == 0:
        pltpu.sync_copy(den_s.at[pl.ds(s * RPT, RPT)], tmp1d_v)
        pltpu.sync_copy(tmp1d_v, den_hbm.at[c, pl.ds(s * RPT, RPT)])

      plsc.subcore_barrier()

  return k


def _project(x, W, a_s, a_d):
  Din = x.shape[1]
  D = W.shape[1]

  def body(x_ref, w_ref, as_ref, ad_ref, ha_ref, hb_ref, pas_ref, pad_ref):
    i = pl.program_id(0)
    h = jnp.dot(x_ref[...], w_ref[...], preferred_element_type=jnp.float32)
    ha_ref[...] = h[:, :DH2]
    hb_ref[...] = h[:, DH2:]
    sl = pl.ds(i * BN_BLK, BN_BLK)
    pas_ref[sl] = jnp.dot(h, as_ref[...], preferred_element_type=jnp.float32)
    pad_ref[sl] = jnp.dot(h, ad_ref[...], preferred_element_type=jnp.float32)

  return pl.pallas_call(
      body,
      grid=(NBLK,),
      in_specs=[
          pl.BlockSpec((BN_BLK, Din), lambda i: (i, 0)),
          pl.BlockSpec((Din, D), lambda i: (0, 0)),
          pl.BlockSpec((D,), lambda i: (0,)),
          pl.BlockSpec((D,), lambda i: (0,)),
      ],
      out_specs=[
          pl.BlockSpec((BN_BLK, DH2), lambda i: (i, 0)),
          pl.BlockSpec((BN_BLK, DH2), lambda i: (i, 0)),
          pl.BlockSpec((NP,), lambda i: (0,)),
          pl.BlockSpec((NP,), lambda i: (0,)),
      ],
      out_shape=[
          jax.ShapeDtypeStruct((NP, DH2), jnp.float32),
          jax.ShapeDtypeStruct((NP, DH2), jnp.float32),
          jax.ShapeDtypeStruct((NP,), jnp.float32),
          jax.ShapeDtypeStruct((NP,), jnp.float32),
      ],
  )(x, W, a_s, a_d)


def _merge(acc, den_t):
  D = acc.shape[2]

  def body(a0_ref, a1_ref, dn_ref, o_ref, st_ref):
    i = pl.program_id(0)
    o = a0_ref[...] + a1_ref[...]
    dn = dn_ref[...]
    dsum = dn[:, 0:1] + dn[:, 1:2]
    o = o / (dsum + 1e-16)
    o_ref[...] = o

    @pl.when(i == 0)
    def _():
      st_ref[...] = jnp.zeros_like(st_ref)

    st_ref[0:1, :] += jnp.sum(o, axis=0, keepdims=True)
    st_ref[1:2, :] += jnp.sum(o * o, axis=0, keepdims=True)

  return pl.pallas_call(
      body,
      grid=(NBLK,),
      in_specs=[
          pl.BlockSpec((BN_BLK, D), lambda i: (i, 0)),
          pl.BlockSpec((BN_BLK, D), lambda i: (i, 0)),
          pl.BlockSpec((BN_BLK, NC), lambda i: (i, 0)),
      ],
      out_specs=[
          pl.BlockSpec((BN_BLK, D), lambda i: (i, 0)),
          pl.BlockSpec((8, D), lambda i: (0, 0)),
      ],
      out_shape=[
          jax.ShapeDtypeStruct((NP, D), jnp.float32),
          jax.ShapeDtypeStruct((8, D), jnp.float32),
      ],
  )(acc[0], acc[1], den_t)


def _merge2(accA, accB, den_t):
  def body(aa0_ref, aa1_ref, ab0_ref, ab1_ref, dn_ref, o_ref, st_ref):
    i = pl.program_id(0)
    dn = dn_ref[...]
    dsum = dn[:, 0:1] + dn[:, 1:2] + 1e-16
    oa = (aa0_ref[...] + aa1_ref[...]) / dsum
    ob = (ab0_ref[...] + ab1_ref[...]) / dsum
    o = jnp.concatenate([oa, ob], axis=1)
    o_ref[...] = o

    @pl.when(i == 0)
    def _():
      st_ref[...] = jnp.zeros_like(st_ref)

    st_ref[0:1, :] += jnp.sum(o, axis=0, keepdims=True)
    st_ref[1:2, :] += jnp.sum(o * o, axis=0, keepdims=True)

  return pl.pallas_call(
      body,
      grid=(NBLK,),
      in_specs=[
          pl.BlockSpec((BN_BLK, DH2), lambda i: (i, 0)),
          pl.BlockSpec((BN_BLK, DH2), lambda i: (i, 0)),
          pl.BlockSpec((BN_BLK, DH2), lambda i: (i, 0)),
          pl.BlockSpec((BN_BLK, DH2), lambda i: (i, 0)),
          pl.BlockSpec((BN_BLK, NC), lambda i: (i, 0)),
      ],
      out_specs=[
          pl.BlockSpec((BN_BLK, 2 * DH2), lambda i: (i, 0)),
          pl.BlockSpec((8, 2 * DH2), lambda i: (0, 0)),
      ],
      out_shape=[
          jax.ShapeDtypeStruct((NP, 2 * DH2), jnp.float32),
          jax.ShapeDtypeStruct((8, 2 * DH2), jnp.float32),
      ],
  )(accA[0], accA[1], accB[0], accB[1], den_t)


def _normproj(o, st, g, be, W, a_s, a_d):
  D = o.shape[1]
  DO = W.shape[1]

  def body(o_ref, st_ref, g_ref, be_ref, w_ref, as_ref, ad_ref,
           h_ref, pas_ref, pad_ref):
    st = st_ref[...]
    mu = st[0:1, :] / N
    var = st[1:2, :] / N - mu * mu
    inv = lax.rsqrt(var + 1e-5)
    xn = g_ref[...][None, :] * (o_ref[...] - mu) * inv + be_ref[...][None, :]
    xr = jnp.maximum(xn, 0.0)
    h = jnp.dot(xr, w_ref[...], preferred_element_type=jnp.float32)
    h_ref[...] = h
    i = pl.program_id(0)
    sl = pl.ds(i * BN_BLK, BN_BLK)
    pas_ref[sl] = jnp.dot(h, as_ref[...], preferred_element_type=jnp.float32)
    pad_ref[sl] = jnp.dot(h, ad_ref[...], preferred_element_type=jnp.float32)

  return pl.pallas_call(
      body,
      grid=(NBLK,),
      in_specs=[
          pl.BlockSpec((BN_BLK, D), lambda i: (i, 0)),
          pl.BlockSpec((8, D), lambda i: (0, 0)),
          pl.BlockSpec((D,), lambda i: (0,)),
          pl.BlockSpec((D,), lambda i: (0,)),
          pl.BlockSpec((D, DO), lambda i: (0, 0)),
          pl.BlockSpec((DO,), lambda i: (0,)),
          pl.BlockSpec((DO,), lambda i: (0,)),
      ],
      out_specs=[
          pl.BlockSpec((BN_BLK, DO), lambda i: (i, 0)),
          pl.BlockSpec((NP,), lambda i: (0,)),
          pl.BlockSpec((NP,), lambda i: (0,)),
      ],
      out_shape=[
          jax.ShapeDtypeStruct((NP, DO), jnp.float32),
          jax.ShapeDtypeStruct((NP,), jnp.float32),
          jax.ShapeDtypeStruct((NP,), jnp.float32),
      ],
  )(o, st, g, be, W, a_s, a_d)


def _tail(o, st, g, be, batch, f1w, f1b, f2w, f2b, f3w, f3b):
  D = o.shape[1]
  DH = f1w.shape[1]

  def body(o_ref, st_ref, g_ref, be_ref, bt_ref,
           f1w_ref, f1b_ref, f2w_ref, f2b_ref, f3w_ref, f3b_ref,
           z_ref, psum, pcnt):
    i = pl.program_id(0)
    st = st_ref[...]
    mu = st[0:1, :] / N
    var = st[1:2, :] / N - mu * mu
    inv = lax.rsqrt(var + 1e-5)
    xn = g_ref[...][None, :] * (o_ref[...] - mu) * inv + be_ref[...][None, :]
    h = jnp.maximum(xn, 0.0)

    bt = bt_ref[pl.ds(i * BN_BLK, BN_BLK)]
    oh = (bt[:, None] == lax.broadcasted_iota(jnp.int32, (1, G), 1)
          ).astype(jnp.float32)

    @pl.when(i == 0)
    def _():
      psum[...] = jnp.zeros_like(psum)
      pcnt[...] = jnp.zeros_like(pcnt)

    psum[...] += lax.dot_general(oh, h, (((0,), (0,)), ((), ())),
                                 preferred_element_type=jnp.float32)
    ones = jnp.ones((BN_BLK, 1), jnp.float32)
    pcnt[...] += lax.dot_general(oh, ones, (((0,), (0,)), ((), ())),
                                 preferred_element_type=jnp.float32)

    @pl.when(i == NBLK - 1)
    def _():
      pooled = psum[...] / jnp.maximum(pcnt[...], 1.0)
      z = jnp.maximum(
          jnp.dot(pooled, f1w_ref[...], preferred_element_type=jnp.float32)
          + f1b_ref[...][None, :], 0.0)
      z = jnp.maximum(
          jnp.dot(z, f2w_ref[...], preferred_element_type=jnp.float32)
          + f2b_ref[...][None, :], 0.0)
      z = (jnp.dot(z, f3w_ref[...], preferred_element_type=jnp.float32)
           + f3b_ref[...][None, :])
      z_ref[...] = z

  return pl.pallas_call(
      body,
      grid=(NBLK,),
      in_specs=[
          pl.BlockSpec((BN_BLK, D), lambda i: (i, 0)),
          pl.BlockSpec((8, D), lambda i: (0, 0)),
          pl.BlockSpec((D,), lambda i: (0,)),
          pl.BlockSpec((D,), lambda i: (0,)),
          pl.BlockSpec((NP,), lambda i: (0,)),
          pl.BlockSpec((D, DH), lambda i: (0, 0)),
          pl.BlockSpec((DH,), lambda i: (0,)),
          pl.BlockSpec((DH, DH // 2), lambda i: (0, 0)),
          pl.BlockSpec((DH // 2,), lambda i: (0,)),
          pl.BlockSpec((DH // 2, 1), lambda i: (0, 0)),
          pl.BlockSpec((1,), lambda i: (0,)),
      ],
      out_specs=pl.BlockSpec((G, 1), lambda i: (0, 0)),
      out_shape=jax.ShapeDtypeStruct((G, 1), jnp.float32),
      scratch_shapes=[
          pltpu.VMEM((G, D), jnp.float32),
          pltpu.VMEM((G, 1), jnp.float32),
      ],
  )(o, st, g, be, batch, f1w, f1b, f2w, f2b, f3w, f3b)


def kernel(x, edge_index, batch, W1, a_src1, a_dst1, b1, g1, be1,
           W2, a_src2, a_dst2, b2, g2, be2,
           fc1_w, fc1_b, fc2_w, fc2_b, fc3_w, fc3_b):
  loop = jnp.arange(N, dtype=edge_index.dtype)
  pad = jnp.zeros((EPAD - ETOT,), dtype=edge_index.dtype)
  src3 = jnp.concatenate([edge_index[0], loop, pad]).reshape(NW, NSUB, K)
  dst3 = jnp.concatenate([edge_index[1], loop, pad]).reshape(NW, NSUB, K)

  x_p = jnp.concatenate(
      [x, jnp.zeros((NP - N, x.shape[1]), x.dtype)], axis=0)
  batch_p = jnp.concatenate(
      [batch, jnp.full((NP - N,), G, batch.dtype)], axis=0)

  gat1 = _gat_edge_call(2)
  gat2 = _gat_edge_call(1)

  h1a, h1b, as1, ad1 = _project(x_p, W1, a_src1, a_dst1)
  acc1, den1 = gat1(h1a, h1b, as1, ad1, src3, dst3)
  o1, st1 = _merge2(acc1[0], acc1[1], den1.T)
  h2, as2, ad2 = _normproj(o1, st1, g1, be1, W2, a_src2, a_dst2)
  acc2, den2 = gat2(h2, as2, ad2, src3, dst3)
  o2, st2 = _merge(acc2[0], den2.T)
  z = _tail(o2, st2, g2, be2, batch_p, fc1_w, fc1_b, fc2_w, fc2_b, fc3_w,
            fc3_b)
  return z.reshape(G)


# speedup vs baseline: 23.7411x; 1.0090x over previous
import dataclasses
import functools

import jax
import jax.numpy as jnp
from jax import lax
from jax.experimental import pallas as pl
from jax.experimental.pallas import tpu as pltpu
from jax.experimental.pallas import tpu_sc as plsc

N = 10000
NP = 10240
E = 320000
G = 64

NC = 2
NS = 16
NW = NC * NS
K = 64
NSUB = 164
CHUNK = NSUB * K
EPAD = NW * CHUNK
ETOT = E + N

RPT = NP // NS

BN_BLK = 1024
NBLK = NP // BN_BLK


DH2 = 64


def _gat_edge_call(nh):
  mesh = plsc.VectorSubcoreMesh(
      core_axis_name="c", subcore_axis_name="s", num_cores=NC, num_subcores=NS)
  cp = pltpu.CompilerParams()
  for fld, val in (("needs_layout_passes", False),
                   ("use_tc_tiling_on_sc", False)):
    if fld in pltpu.CompilerParams.__dataclass_fields__:
      cp = dataclasses.replace(cp, **{fld: val})

  @functools.partial(
      pl.kernel,
      compiler_params=cp,
      out_type=(jax.ShapeDtypeStruct((nh, NC, NP, DH2), jnp.float32),
                jax.ShapeDtypeStruct((NC, NP), jnp.float32)),
      mesh=mesh,
      scratch_types=[
          pltpu.VMEM((NP,), jnp.float32),
          pltpu.VMEM((NP,), jnp.float32),
          pltpu.VMEM((NSUB, K), jnp.int32),
          pltpu.VMEM((NSUB, K), jnp.int32),
          pltpu.VMEM((NSUB, K), jnp.float32),
          pltpu.VMEM((K, DH2), jnp.float32),
          pltpu.VMEM((K, DH2), jnp.float32),
          pltpu.VMEM((RPT,), jnp.float32),
          pltpu.VMEM_SHARED((NP, DH2), jnp.float32),
          pltpu.VMEM_SHARED((NP,), jnp.float32),
          pltpu.SemaphoreType.DMA,
          pltpu.SemaphoreType.DMA,
          pltpu.SemaphoreType.DMA,
          pltpu.SemaphoreType.DMA,
          pltpu.SemaphoreType.DMA,
      ],
  )
  def k(*refs):
    h_hbms = refs[0:nh]
    asrc_hbm, adst_hbm, src_hbm, dst_hbm = refs[nh:nh + 4]
    acc_hbm, den_hbm = refs[nh + 4:nh + 6]
    (asrc_v, adst_v, src_v, dst_v, ex_v, rows0, rows1, tmp1d_v,
     acc_s, den_s, g0, g1, s0, s1, dsem) = refs[nh + 6:]

    c = lax.axis_index("c")
    s = lax.axis_index("s")
    wid = c * NS + s
    ebase = wid * CHUNK
    zf = jnp.zeros((16,), jnp.float32)
    rows = (rows0, rows1)
    gsem = (g0, g1)
    ssem = (s0, s1)

    pltpu.async_copy(asrc_hbm, asrc_v, g0)
    pltpu.async_copy(adst_hbm, adst_v, g1)
    pltpu.async_copy(src_hbm.at[wid], src_v, s0)
    pltpu.async_copy(dst_hbm.at[wid], dst_v, s1)

    @pl.loop(0, RPT // 16)
    def _(t):
      tmp1d_v[pl.ds(t * 16, 16)] = zf

    pltpu.make_async_copy(asrc_hbm, asrc_v, g0).wait()
    pltpu.make_async_copy(adst_hbm, adst_v, g1).wait()
    pltpu.make_async_copy(src_hbm.at[wid], src_v, s0).wait()
    pltpu.make_async_copy(dst_hbm.at[wid], dst_v, s1).wait()

    def prep(j, hf):
      if hf == 0:
        for i in range(K // 16):
          sl = pl.ds(i * 16, 16)
          sv = src_v[j, sl]
          dv = dst_v[j, sl]
          vs = plsc.load_gather(asrc_v, [sv])
          vd = plsc.load_gather(adst_v, [dv])
          e = vs + vd
          e = jnp.where(e >= 0.0, e, 0.2 * e)
          ex = jnp.exp(e)
          gi = ebase + j * K + i * 16 + lax.iota(jnp.int32, 16)
          ex = jnp.where(gi < ETOT, ex, 0.0)
          ex_v[j, sl] = ex
        pltpu.async_copy(ex_v.at[j], den_s.at[dst_v.at[j]], dsem, add=True)

    def gather_start(j, p, hf):
      pltpu.async_copy(h_hbms[hf].at[src_v.at[j]], rows[p], gsem[p])

    def gather_wait(j, p, hf):
      pltpu.make_async_copy(
          h_hbms[hf].at[src_v.at[j]], rows[p], gsem[p]).wait()

    def scale(j, p):
      @pl.loop(0, K, unroll=8)
      def _(i):
        jv = jnp.full((16,), j, jnp.int32)
        iv = jnp.full((16,), i, jnp.int32)
        w = plsc.load_gather(ex_v, [jv, iv])
        for q in range(DH2 // 16):
          sl = pl.ds(q * 16, 16)
          rows[p][i, sl] = rows[p][i, sl] * w

    def scatter_start(j, p):
      pltpu.async_copy(rows[p], acc_s.at[dst_v.at[j]], ssem[p], add=True)

    def scatter_wait(j, p):
      pltpu.make_async_copy(
          rows[p], acc_s.at[dst_v.at[j]], ssem[p]).wait()

    for hf in range(nh):
      @pl.loop(0, K)
      def _(r):
        for q in range(DH2 // 16):
          rows0[r, pl.ds(q * 16, 16)] = zf

      for q in range(RPT // K):
        pltpu.async_copy(rows0, acc_s.at[pl.ds(s * RPT + q * K, K)], dsem)
      if hf == 0:
        pltpu.async_copy(tmp1d_v, den_s.at[pl.ds(s * RPT, RPT)], dsem)
      for q in range(RPT // K):
        pltpu.make_async_copy(
            rows0, acc_s.at[pl.ds(s * RPT + q * K, K)], dsem).wait()
      if hf == 0:
        pltpu.make_async_copy(
            tmp1d_v, den_s.at[pl.ds(s * RPT, RPT)], dsem).wait()

      plsc.subcore_barrier()

      prep(0, hf)
      gather_start(0, 0, hf)

      @pl.loop(0, NSUB // 2)
      def _(t):
        j0 = 2 * t
        j1 = j0 + 1

        @pl.when(t > 0)
        def _():
          scatter_wait(j1 - 2, 1)
        prep(j1, hf)
        gather_start(j1, 1, hf)

        gather_wait(j0, 0, hf)
        scale(j0, 0)
        scatter_start(j0, 0)

        @pl.when(t < NSUB // 2 - 1)
        def _():
          scatter_wait(j0, 0)
          prep(j0 + 2, hf)
          gather_start(j0 + 2, 0, hf)

        gather_wait(j1, 1, hf)
        scale(j1, 1)
        scatter_start(j1, 1)

      scatter_wait(NSUB - 2, 0)
      scatter_wait(NSUB - 1, 1)
      if hf == 0:
        @pl.loop(0, NSUB)
        def _(t):
          pltpu.make_async_copy(
              ex_v.at[0], den_s.at[dst_v.at[0]], dsem).wait()

      plsc.subcore_barrier()

      if hf == 0:
        pltpu.sync_copy(den_s.at[pl.ds(s * RPT, RPT)], tmp1d_v)
        pltpu.async_copy(tmp1d_v, den_hbm.at[c, pl.ds(s * RPT, RPT)], dsem)
      for q in range(RPT // K):
        r0 = s * RPT + q * K
        p = q % 2
        if q >= 2:
          r0p = s * RPT + (q - 2) * K
          pltpu.make_async_copy(
              rows[p], acc_hbm.at[hf, c, pl.ds(r0p, K)], gsem[p]).wait()
        pltpu.sync_copy(acc_s.at[pl.ds(r0, K)], rows[p])
        pltpu.async_copy(rows[p], acc_hbm.at[hf, c, pl.ds(r0, K)], gsem[p])
      for q in range(RPT // K - 2, RPT // K):
        r0 = s * RPT + q * K
        pltpu.make_async_copy(
            rows[q % 2], acc_hbm.at[hf, c, pl.ds(r0, K)], gsem[q % 2]).wait()
      if hf == 0:
        pltpu.make_async_copy(
            tmp1d_v, den_hbm.at[c, pl.ds(s * RPT, RPT)], dsem).wait()

      plsc.subcore_barrier()

  return k


def _project(x, W, a_s, a_d):
  Din = x.shape[1]
  D = W.shape[1]

  def body(x_ref, w_ref, as_ref, ad_ref, ha_ref, hb_ref, pas_ref, pad_ref):
    i = pl.program_id(0)
    h = jnp.dot(x_ref[...], w_ref[...], preferred_element_type=jnp.float32)
    ha_ref[...] = h[:, :DH2]
    hb_ref[...] = h[:, DH2:]
    sl = pl.ds(i * BN_BLK, BN_BLK)
    pas_ref[sl] = jnp.dot(h, as_ref[...], preferred_element_type=jnp.float32)
    pad_ref[sl] = jnp.dot(h, ad_ref[...], preferred_element_type=jnp.float32)

  return pl.pallas_call(
      body,
      grid=(NBLK,),
      in_specs=[
          pl.BlockSpec((BN_BLK, Din), lambda i: (i, 0)),
          pl.BlockSpec((Din, D), lambda i: (0, 0)),
          pl.BlockSpec((D,), lambda i: (0,)),
          pl.BlockSpec((D,), lambda i: (0,)),
      ],
      out_specs=[
          pl.BlockSpec((BN_BLK, DH2), lambda i: (i, 0)),
          pl.BlockSpec((BN_BLK, DH2), lambda i: (i, 0)),
          pl.BlockSpec((NP,), lambda i: (0,)),
          pl.BlockSpec((NP,), lambda i: (0,)),
      ],
      out_shape=[
          jax.ShapeDtypeStruct((NP, DH2), jnp.float32),
          jax.ShapeDtypeStruct((NP, DH2), jnp.float32),
          jax.ShapeDtypeStruct((NP,), jnp.float32),
          jax.ShapeDtypeStruct((NP,), jnp.float32),
      ],
  )(x, W, a_s, a_d)


def _merge(acc, den_t):
  D = acc.shape[2]

  def body(a0_ref, a1_ref, dn_ref, o_ref, st_ref):
    i = pl.program_id(0)
    o = a0_ref[...] + a1_ref[...]
    dn = dn_ref[...]
    dsum = dn[:, 0:1] + dn[:, 1:2]
    o = o / (dsum + 1e-16)
    o_ref[...] = o

    @pl.when(i == 0)
    def _():
      st_ref[...] = jnp.zeros_like(st_ref)

    st_ref[0:1, :] += jnp.sum(o, axis=0, keepdims=True)
    st_ref[1:2, :] += jnp.sum(o * o, axis=0, keepdims=True)

  return pl.pallas_call(
      body,
      grid=(NBLK,),
      in_specs=[
          pl.BlockSpec((BN_BLK, D), lambda i: (i, 0)),
          pl.BlockSpec((BN_BLK, D), lambda i: (i, 0)),
          pl.BlockSpec((BN_BLK, NC), lambda i: (i, 0)),
      ],
      out_specs=[
          pl.BlockSpec((BN_BLK, D), lambda i: (i, 0)),
          pl.BlockSpec((8, D), lambda i: (0, 0)),
      ],
      out_shape=[
          jax.ShapeDtypeStruct((NP, D), jnp.float32),
          jax.ShapeDtypeStruct((8, D), jnp.float32),
      ],
  )(acc[0], acc[1], den_t)


def _merge2(accA, accB, den_t):
  def body(aa0_ref, aa1_ref, ab0_ref, ab1_ref, dn_ref, o_ref, st_ref):
    i = pl.program_id(0)
    dn = dn_ref[...]
    dsum = dn[:, 0:1] + dn[:, 1:2] + 1e-16
    oa = (aa0_ref[...] + aa1_ref[...]) / dsum
    ob = (ab0_ref[...] + ab1_ref[...]) / dsum
    o = jnp.concatenate([oa, ob], axis=1)
    o_ref[...] = o

    @pl.when(i == 0)
    def _():
      st_ref[...] = jnp.zeros_like(st_ref)

    st_ref[0:1, :] += jnp.sum(o, axis=0, keepdims=True)
    st_ref[1:2, :] += jnp.sum(o * o, axis=0, keepdims=True)

  return pl.pallas_call(
      body,
      grid=(NBLK,),
      in_specs=[
          pl.BlockSpec((BN_BLK, DH2), lambda i: (i, 0)),
          pl.BlockSpec((BN_BLK, DH2), lambda i: (i, 0)),
          pl.BlockSpec((BN_BLK, DH2), lambda i: (i, 0)),
          pl.BlockSpec((BN_BLK, DH2), lambda i: (i, 0)),
          pl.BlockSpec((BN_BLK, NC), lambda i: (i, 0)),
      ],
      out_specs=[
          pl.BlockSpec((BN_BLK, 2 * DH2), lambda i: (i, 0)),
          pl.BlockSpec((8, 2 * DH2), lambda i: (0, 0)),
      ],
      out_shape=[
          jax.ShapeDtypeStruct((NP, 2 * DH2), jnp.float32),
          jax.ShapeDtypeStruct((8, 2 * DH2), jnp.float32),
      ],
  )(accA[0], accA[1], accB[0], accB[1], den_t)


def _normproj(o, st, g, be, W, a_s, a_d):
  D = o.shape[1]
  DO = W.shape[1]

  def body(o_ref, st_ref, g_ref, be_ref, w_ref, as_ref, ad_ref,
           h_ref, pas_ref, pad_ref):
    st = st_ref[...]
    mu = st[0:1, :] / N
    var = st[1:2, :] / N - mu * mu
    inv = lax.rsqrt(var + 1e-5)
    xn = g_ref[...][None, :] * (o_ref[...] - mu) * inv + be_ref[...][None, :]
    xr = jnp.maximum(xn, 0.0)
    h = jnp.dot(xr, w_ref[...], preferred_element_type=jnp.float32)
    h_ref[...] = h
    i = pl.program_id(0)
    sl = pl.ds(i * BN_BLK, BN_BLK)
    pas_ref[sl] = jnp.dot(h, as_ref[...], preferred_element_type=jnp.float32)
    pad_ref[sl] = jnp.dot(h, ad_ref[...], preferred_element_type=jnp.float32)

  return pl.pallas_call(
      body,
      grid=(NBLK,),
      in_specs=[
          pl.BlockSpec((BN_BLK, D), lambda i: (i, 0)),
          pl.BlockSpec((8, D), lambda i: (0, 0)),
          pl.BlockSpec((D,), lambda i: (0,)),
          pl.BlockSpec((D,), lambda i: (0,)),
          pl.BlockSpec((D, DO), lambda i: (0, 0)),
          pl.BlockSpec((DO,), lambda i: (0,)),
          pl.BlockSpec((DO,), lambda i: (0,)),
      ],
      out_specs=[
          pl.BlockSpec((BN_BLK, DO), lambda i: (i, 0)),
          pl.BlockSpec((NP,), lambda i: (0,)),
          pl.BlockSpec((NP,), lambda i: (0,)),
      ],
      out_shape=[
          jax.ShapeDtypeStruct((NP, DO), jnp.float32),
          jax.ShapeDtypeStruct((NP,), jnp.float32),
          jax.ShapeDtypeStruct((NP,), jnp.float32),
      ],
  )(o, st, g, be, W, a_s, a_d)


def _tail(o, st, g, be, batch, f1w, f1b, f2w, f2b, f3w, f3b):
  D = o.shape[1]
  DH = f1w.shape[1]

  def body(o_ref, st_ref, g_ref, be_ref, bt_ref,
           f1w_ref, f1b_ref, f2w_ref, f2b_ref, f3w_ref, f3b_ref,
           z_ref, psum, pcnt):
    i = pl.program_id(0)
    st = st_ref[...]
    mu = st[0:1, :] / N
    var = st[1:2, :] / N - mu * mu
    inv = lax.rsqrt(var + 1e-5)
    xn = g_ref[...][None, :] * (o_ref[...] - mu) * inv + be_ref[...][None, :]
    h = jnp.maximum(xn, 0.0)

    bt = bt_ref[pl.ds(i * BN_BLK, BN_BLK)]
    oh = (bt[:, None] == lax.broadcasted_iota(jnp.int32, (1, G), 1)
          ).astype(jnp.float32)

    @pl.when(i == 0)
    def _():
      psum[...] = jnp.zeros_like(psum)
      pcnt[...] = jnp.zeros_like(pcnt)

    psum[...] += lax.dot_general(oh, h, (((0,), (0,)), ((), ())),
                                 preferred_element_type=jnp.float32)
    ones = jnp.ones((BN_BLK, 1), jnp.float32)
    pcnt[...] += lax.dot_general(oh, ones, (((0,), (0,)), ((), ())),
                                 preferred_element_type=jnp.float32)

    @pl.when(i == NBLK - 1)
    def _():
      pooled = psum[...] / jnp.maximum(pcnt[...], 1.0)
      z = jnp.maximum(
          jnp.dot(pooled, f1w_ref[...], preferred_element_type=jnp.float32)
          + f1b_ref[...][None, :], 0.0)
      z = jnp.maximum(
          jnp.dot(z, f2w_ref[...], preferred_element_type=jnp.float32)
          + f2b_ref[...][None, :], 0.0)
      z = (jnp.dot(z, f3w_ref[...], preferred_element_type=jnp.float32)
           + f3b_ref[...][None, :])
      z_ref[...] = z

  return pl.pallas_call(
      body,
      grid=(NBLK,),
      in_specs=[
          pl.BlockSpec((BN_BLK, D), lambda i: (i, 0)),
          pl.BlockSpec((8, D), lambda i: (0, 0)),
          pl.BlockSpec((D,), lambda i: (0,)),
          pl.BlockSpec((D,), lambda i: (0,)),
          pl.BlockSpec((NP,), lambda i: (0,)),
          pl.BlockSpec((D, DH), lambda i: (0, 0)),
          pl.BlockSpec((DH,), lambda i: (0,)),
          pl.BlockSpec((DH, DH // 2), lambda i: (0, 0)),
          pl.BlockSpec((DH // 2,), lambda i: (0,)),
          pl.BlockSpec((DH // 2, 1), lambda i: (0, 0)),
          pl.BlockSpec((1,), lambda i: (0,)),
      ],
      out_specs=pl.BlockSpec((G, 1), lambda i: (0, 0)),
      out_shape=jax.ShapeDtypeStruct((G, 1), jnp.float32),
      scratch_shapes=[
          pltpu.VMEM((G, D), jnp.float32),
          pltpu.VMEM((G, 1), jnp.float32),
      ],
  )(o, st, g, be, batch, f1w, f1b, f2w, f2b, f3w, f3b)


def kernel(x, edge_index, batch, W1, a_src1, a_dst1, b1, g1, be1,
           W2, a_src2, a_dst2, b2, g2, be2,
           fc1_w, fc1_b, fc2_w, fc2_b, fc3_w, fc3_b):
  loop = jnp.arange(N, dtype=edge_index.dtype)
  pad = jnp.zeros((EPAD - ETOT,), dtype=edge_index.dtype)
  src3 = jnp.concatenate([edge_index[0], loop, pad]).reshape(NW, NSUB, K)
  dst3 = jnp.concatenate([edge_index[1], loop, pad]).reshape(NW, NSUB, K)

  x_p = jnp.concatenate(
      [x, jnp.zeros((NP - N, x.shape[1]), x.dtype)], axis=0)
  batch_p = jnp.concatenate(
      [batch, jnp.full((NP - N,), G, batch.dtype)], axis=0)

  gat1 = _gat_edge_call(2)
  gat2 = _gat_edge_call(1)

  h1a, h1b, as1, ad1 = _project(x_p, W1, a_src1, a_dst1)
  acc1, den1 = gat1(h1a, h1b, as1, ad1, src3, dst3)
  o1, st1 = _merge2(acc1[0], acc1[1], den1.T)
  h2, as2, ad2 = _normproj(o1, st1, g1, be1, W2, a_src2, a_dst2)
  acc2, den2 = gat2(h2, as2, ad2, src3, dst3)
  o2, st2 = _merge(acc2[0], den2.T)
  z = _tail(o2, st2, g2, be2, batch_p, fc1_w, fc1_b, fc2_w, fc2_b, fc3_w,
            fc3_b)
  return z.reshape(G)
